# baseline (speedup 1.0000x reference)
import functools

import jax
import jax.numpy as jnp
from jax import lax
from jax.experimental import pallas as pl
from jax.experimental.pallas import tpu as pltpu
from jax.experimental.pallas import tpu_sc as plsc

NUM_EMBEDDINGS = 100000
D = 768
CONTEXT_LENGTH = 2048
BATCH = 4
B_TOTAL = BATCH * CONTEXT_LENGTH

NC, NS = 2, 16
NW = NC * NS
POS_PER_W = CONTEXT_LENGTH // NW
CHUNK = 32
HALVES = POS_PER_W // CHUNK
N_CHUNKS = BATCH * HALVES
LANES = 16
VECS_PER_ROW = D // LANES


NBUF = 3


def _embed_body(x_hbm, tok_hbm, pos_hbm, out_hbm, idx_v, rows_v, pos_v,
                sem_g0, sem_g1, sem_g2, sem_o0, sem_o1, sem_o2, sem_p0):
    wid = lax.axis_index("s") * NC + lax.axis_index("c")
    p0 = wid * POS_PER_W

    sem_g = (sem_g0, sem_g1, sem_g2)
    sem_o = (sem_o0, sem_o1, sem_o2)

    for b in range(BATCH):
        pltpu.sync_copy(x_hbm.at[b, pl.ds(p0, POS_PER_W)], idx_v.at[b])
    p_stage = pltpu.async_copy(
        pos_hbm.at[pl.ds(p0, POS_PER_W)], pos_v, sem_p0)

    def gather(c):
        b, h = divmod(c, HALVES)
        return pltpu.async_copy(
            tok_hbm.at[idx_v.at[b, pl.ds(h * CHUNK, CHUNK)]],
            rows_v.at[c % NBUF], sem_g[c % NBUF])

    copies = {0: gather(0), 1: gather(1)}
    p_stage.wait()
    out_copies = {}
    for c in range(N_CHUNKS):
        b, h = divmod(c, HALVES)
        if c + 2 < N_CHUNKS:
            if c - 1 >= 0:
                out_copies[c - 1].wait()
            copies[c + 2] = gather(c + 2)
        copies[c].wait()

        buf = rows_v.at[c % NBUF]
        ph = h * CHUNK

        @plsc.parallel_loop(0, CHUNK, step=1, unroll=4)
        def _add_row(r):
            for v in range(VECS_PER_ROW):
                sl = pl.ds(v * LANES, LANES)
                plsc.addupdate(buf.at[r, sl], pos_v[ph + r, sl])

        row0 = b * CONTEXT_LENGTH + p0 + ph
        out_copies[c] = pltpu.async_copy(
            buf, out_hbm.at[pl.ds(row0, CHUNK)], sem_o[c % NBUF])
    for c in range(max(0, N_CHUNKS - 3), N_CHUNKS):
        out_copies[c].wait()


@jax.jit
def _embed(x_grouped, tok_emb_weight, pos_emb_weight):
    mesh = plsc.VectorSubcoreMesh(
        core_axis_name="c", subcore_axis_name="s", num_cores=NC,
        num_subcores=NS)
    return pl.kernel(
        _embed_body,
        out_type=jax.ShapeDtypeStruct((B_TOTAL, D), jnp.float32),
        mesh=mesh,
        scratch_types=[
            pltpu.VMEM((BATCH, POS_PER_W), jnp.int32),
            pltpu.VMEM((NBUF, CHUNK, D), jnp.float32),
            pltpu.VMEM((POS_PER_W, D), jnp.float32),
            pltpu.SemaphoreType.DMA,
            pltpu.SemaphoreType.DMA,
            pltpu.SemaphoreType.DMA,
            pltpu.SemaphoreType.DMA,
            pltpu.SemaphoreType.DMA,
            pltpu.SemaphoreType.DMA,
            pltpu.SemaphoreType.DMA,
        ],
    )(x_grouped, tok_emb_weight, pos_emb_weight)


def kernel(x, tok_emb_weight, pos_emb_weight):
    batch, cxt = x.shape
    out = _embed(x.astype(jnp.int32), tok_emb_weight, pos_emb_weight)
    return out.reshape(batch, cxt, D)


# speedup vs baseline: 1.1325x; 1.1325x over previous
import jax
import jax.numpy as jnp
from jax import lax
from jax.experimental import pallas as pl
from jax.experimental.pallas import tpu as pltpu
from jax.experimental.pallas import tpu_sc as plsc

NUM_EMBEDDINGS = 100000
D = 768
CONTEXT_LENGTH = 2048
BATCH = 4
B_TOTAL = BATCH * CONTEXT_LENGTH

NC, NS = 2, 16
NW = NC * NS
POS_PER_W = CONTEXT_LENGTH // NW
CHUNK = 16
QUARTERS = POS_PER_W // CHUNK
N_CHUNKS = BATCH * QUARTERS
LANES = 16
VECS_PER_ROW = D // LANES
NBUF = 6
DEPTH = 3


def _embed_body(x_hbm, tok_hbm, pos_hbm, out_hbm, idx_v, rows_v, pos_v,
                *sems):
    wid = lax.axis_index("s") * NC + lax.axis_index("c")
    p0 = wid * POS_PER_W

    sem_g = sems[:NBUF]
    sem_o = sems[NBUF:2 * NBUF]
    sem_p = sems[2 * NBUF]

    for b in range(BATCH):
        pltpu.sync_copy(x_hbm.at[b, pl.ds(p0, POS_PER_W)], idx_v.at[b])
    p_stage = pltpu.async_copy(
        pos_hbm.at[pl.ds(p0, POS_PER_W)], pos_v, sem_p)

    def gather(c):
        b, q = divmod(c, QUARTERS)
        return pltpu.async_copy(
            tok_hbm.at[idx_v.at[b, pl.ds(q * CHUNK, CHUNK)]],
            rows_v.at[c % NBUF], sem_g[c % NBUF])

    copies = {}
    for k in range(DEPTH):
        copies[k] = gather(k)
    p_stage.wait()

    out_copies = {}
    for c in range(N_CHUNKS):
        b, q = divmod(c, QUARTERS)
        if c + DEPTH < N_CHUNKS:
            if c + DEPTH - NBUF >= 0:
                out_copies[c + DEPTH - NBUF].wait()
            copies[c + DEPTH] = gather(c + DEPTH)
        copies[c].wait()

        buf = rows_v.at[c % NBUF]
        ph = q * CHUNK

        @plsc.parallel_loop(0, CHUNK, step=1, unroll=1)
        def _add_row(r):
            for v in range(VECS_PER_ROW):
                sl = pl.ds(v * LANES, LANES)
                plsc.addupdate(buf.at[r, sl], pos_v[ph + r, sl])

        row0 = b * CONTEXT_LENGTH + p0 + ph
        out_copies[c] = pltpu.async_copy(
            buf, out_hbm.at[pl.ds(row0, CHUNK)], sem_o[c % NBUF])
    for c in range(N_CHUNKS - NBUF, N_CHUNKS):
        out_copies[c].wait()


@jax.jit
def _embed(x_grouped, tok_emb_weight, pos_emb_weight):
    mesh = plsc.VectorSubcoreMesh(
        core_axis_name="c", subcore_axis_name="s", num_cores=NC,
        num_subcores=NS)
    return pl.kernel(
        _embed_body,
        out_type=jax.ShapeDtypeStruct((B_TOTAL, D), jnp.float32),
        mesh=mesh,
        scratch_types=[
            pltpu.VMEM((BATCH, POS_PER_W), jnp.int32),
            pltpu.VMEM((NBUF, CHUNK, D), jnp.float32),
            pltpu.VMEM((POS_PER_W, D), jnp.float32),
        ] + [pltpu.SemaphoreType.DMA] * (2 * NBUF + 1),
    )(x_grouped, tok_emb_weight, pos_emb_weight)


def kernel(x, tok_emb_weight, pos_emb_weight):
    batch, cxt = x.shape
    out = _embed(x.astype(jnp.int32), tok_emb_weight, pos_emb_weight)
    return out.reshape(batch, cxt, D)


# speedup vs baseline: 1.3885x; 1.2261x over previous
import jax
import jax.numpy as jnp
from jax import lax
from jax.experimental import pallas as pl
from jax.experimental.pallas import tpu as pltpu
from jax.experimental.pallas import tpu_sc as plsc

NUM_EMBEDDINGS = 100000
D = 768
CONTEXT_LENGTH = 2048
BATCH = 4
B_TOTAL = BATCH * CONTEXT_LENGTH

NC, NS = 2, 16
NW = NC * NS
POS_PER_W = CONTEXT_LENGTH // NW
CHUNK = 16
QUARTERS = POS_PER_W // CHUNK
N_CHUNKS = BATCH * QUARTERS
LANES = 16
VECS_PER_ROW = D // LANES
NBUF = 6
DEPTH = 4


def _embed_body(x_hbm, tok_hbm, pos_hbm, out_hbm, idx_v, rows_v, pos_v,
                *sems):
    wid = lax.axis_index("s") * NC + lax.axis_index("c")
    p0 = wid * POS_PER_W

    sem_g = sems[:NBUF]
    sem_o = sems[NBUF:2 * NBUF]
    sem_p = sems[2 * NBUF]
    sem_i = sems[2 * NBUF + 1]

    idx_copies = [
        pltpu.async_copy(x_hbm.at[b, pl.ds(p0, POS_PER_W)], idx_v.at[b],
                         sem_i)
        for b in range(BATCH)
    ]
    p_stage = pltpu.async_copy(
        pos_hbm.at[pl.ds(p0, POS_PER_W)], pos_v, sem_p)
    for cp in idx_copies:
        cp.wait()

    def gather(c):
        b, q = divmod(c, QUARTERS)
        return pltpu.async_copy(
            tok_hbm.at[idx_v.at[b, pl.ds(q * CHUNK, CHUNK)]],
            rows_v.at[c % NBUF], sem_g[c % NBUF])

    copies = {}
    for k in range(DEPTH):
        copies[k] = gather(k)
    p_stage.wait()

    out_copies = {}
    for c in range(N_CHUNKS):
        b, q = divmod(c, QUARTERS)
        if c + DEPTH < N_CHUNKS:
            if c + DEPTH - NBUF >= 0:
                out_copies[c + DEPTH - NBUF].wait()
            copies[c + DEPTH] = gather(c + DEPTH)
        copies[c].wait()

        buf = rows_v.at[c % NBUF]
        ph = q * CHUNK

        @plsc.parallel_loop(0, VECS_PER_ROW, step=1, unroll=1)
        def _add_col(v):
            sl = pl.ds(v * LANES, LANES)
            for r in range(CHUNK):
                plsc.addupdate(buf.at[r, sl], pos_v[ph + r, sl])

        row0 = b * CONTEXT_LENGTH + p0 + ph
        out_copies[c] = pltpu.async_copy(
            buf, out_hbm.at[pl.ds(row0, CHUNK)], sem_o[c % NBUF])
    for c in range(N_CHUNKS - NBUF, N_CHUNKS):
        out_copies[c].wait()


@jax.jit
def _embed(x_grouped, tok_emb_weight, pos_emb_weight):
    mesh = plsc.VectorSubcoreMesh(
        core_axis_name="c", subcore_axis_name="s", num_cores=NC,
        num_subcores=NS)
    return pl.kernel(
        _embed_body,
        out_type=jax.ShapeDtypeStruct((B_TOTAL, D), jnp.float32),
        mesh=mesh,
        scratch_types=[
            pltpu.VMEM((BATCH, POS_PER_W), jnp.int32),
            pltpu.VMEM((NBUF, CHUNK, D), jnp.float32),
            pltpu.VMEM((POS_PER_W, D), jnp.float32),
        ] + [pltpu.SemaphoreType.DMA] * (2 * NBUF + 2),
    )(x_grouped, tok_emb_weight, pos_emb_weight)


def kernel(x, tok_emb_weight, pos_emb_weight):
    batch, cxt = x.shape
    out = _embed(x.astype(jnp.int32), tok_emb_weight, pos_emb_weight)
    return out.reshape(batch, cxt, D)
